# baseline (speedup 1.0000x reference)
import jax
import jax.numpy as jnp
from jax import lax
from jax.experimental import pallas as pl
from jax.experimental.pallas import tpu as pltpu
from jax.experimental.pallas import tpu_sc as plsc

N, K, D = 10000, 32, 128

NC, NS = 2, 16
NW = NC * NS
NPAD = 10240
NPW = 248
S = NW * NPW
M = NPAD - S
G = 2
GK = G * K
NG = NPW // G
NBUF = 4
LANES = 16
DV = D // LANES

MM_BLOCK = 1024
CHUNK = 256


def _mm_body(f_ref, w_ref, b_ref, o_ref):
    prod = lax.dot_general(f_ref[...], w_ref[...], (((1,), (1,)), ((), ())),
                           preferred_element_type=jnp.float32)
    o_ref[...] = jnp.maximum(prod + b_ref[...], 0.0)


_mm = pl.pallas_call(
    _mm_body,
    grid=(NPAD // MM_BLOCK,),
    in_specs=[
        pl.BlockSpec((MM_BLOCK, D), lambda i: (i, 0)),
        pl.BlockSpec((D, D), lambda i: (0, 0)),
        pl.BlockSpec((1, D), lambda i: (0, 0)),
    ],
    out_specs=pl.BlockSpec((MM_BLOCK, D), lambda i: (i, 0)),
    out_shape=jax.ShapeDtypeStruct((NPAD, D), jnp.float32),
)


def _sc_body(x_hbm, edge_hbm, out_hbm, idx_v, rows_v, out_v,
             sem0, sem1, sem2, sem3):
    sems = (sem0, sem1, sem2, sem3)
    wid = lax.axis_index("s") * NC + lax.axis_index("c")
    base = wid * NPW
    pltpu.sync_copy(edge_hbm.at[pl.ds(base * K, NPW * K)], idx_v)

    def _gather(g, slot):
        pltpu.async_copy(
            x_hbm.at[idx_v.at[pl.ds(g * GK, GK)]],
            rows_v.at[slot], sems[slot])

    for b in range(NBUF):
        _gather(b, b)

    @pl.loop(0, NG, step=NBUF)
    def _outer(g0):
        for b in range(NBUF):
            g = g0 + b
            pltpu.make_async_copy(
                x_hbm.at[idx_v.at[pl.ds(g * GK, GK)]],
                rows_v.at[b], sems[b]).wait()
            rows = rows_v.at[b]
            for gi in range(G):
                node = g * G + gi
                sls = [pl.ds(dv * LANES, LANES) for dv in range(DV)]
                accs = [rows[gi * K, sl] for sl in sls]
                for k in range(1, K):
                    r = gi * K + k
                    accs = [acc + rows[r, sl] for acc, sl in zip(accs, sls)]
                for dv in range(DV):
                    out_v[node, sls[dv]] = accs[dv] * (1.0 / K)
            gn = g + NBUF

            @pl.when(gn < NG)
            def _():
                _gather(gn, b)

    pltpu.sync_copy(out_v, out_hbm.at[pl.ds(base, NPW)])


_sc_gather_mean = pl.kernel(
    _sc_body,
    out_type=jax.ShapeDtypeStruct((S, D), jnp.float32),
    mesh=plsc.VectorSubcoreMesh(core_axis_name="c", subcore_axis_name="s"),
    scratch_types=[
        pltpu.VMEM((NPW * K,), jnp.int32),
        pltpu.VMEM((NBUF, GK, D), jnp.float32),
        pltpu.VMEM((NPW, D), jnp.float32),
        pltpu.SemaphoreType.DMA,
        pltpu.SemaphoreType.DMA,
        pltpu.SemaphoreType.DMA,
        pltpu.SemaphoreType.DMA,
    ],
)


def _tc_gather_body(e_ref, x_ref, o_ref):
    def node_body(n, carry):
        rows = [x_ref[pl.ds(e_ref[n, k], 1), :] for k in range(K)]
        while len(rows) > 1:
            rows = [rows[i] + rows[i + 1] for i in range(0, len(rows), 2)]
        o_ref[pl.ds(n, 1), :] = rows[0] * (1.0 / K)
        return carry

    lax.fori_loop(0, CHUNK, node_body, 0)


_tc_gather_mean = pl.pallas_call(
    _tc_gather_body,
    grid=(M // CHUNK,),
    in_specs=[
        pl.BlockSpec((CHUNK, K), lambda i: (i, 0),
                     memory_space=pltpu.SMEM),
        pl.BlockSpec((NPAD, D), lambda i: (0, 0)),
    ],
    out_specs=pl.BlockSpec((CHUNK, D), lambda i: (i, 0)),
    out_shape=jax.ShapeDtypeStruct((M, D), jnp.float32),
)


def kernel(idx, feats, edge_dict, sadj, epoch, W, b):
    feats_pad = jnp.concatenate(
        [feats, jnp.zeros((NPAD - N, D), jnp.float32)])
    x = _mm(feats_pad, W, b.reshape(1, D))
    edge = jnp.concatenate(
        [edge_dict.astype(jnp.int32),
         jnp.zeros((NPAD - N, K), jnp.int32)])
    sc_out = _sc_gather_mean(x, edge[:S].reshape(-1))
    tc_out = _tc_gather_mean(edge[S:], x)
    return jnp.concatenate([sc_out, tc_out])[:N]


# speedup vs baseline: 1.0458x; 1.0458x over previous
import jax
import jax.numpy as jnp
from jax import lax
from jax.experimental import pallas as pl
from jax.experimental.pallas import tpu as pltpu
from jax.experimental.pallas import tpu_sc as plsc

N, K, D = 10000, 32, 128

NC, NS = 2, 16
NW = NC * NS
NPAD = 10240
NPW = 232
S = NW * NPW
M = NPAD - S
G = 2
GK = G * K
NG = NPW // G
NBUF = 4
LANES = 16
DV = D // LANES

MM_BLOCK = 1024
CHUNK = 256


def _mm_body(f_ref, w_ref, b_ref, o_ref):
    prod = lax.dot_general(f_ref[...], w_ref[...], (((1,), (1,)), ((), ())),
                           preferred_element_type=jnp.float32)
    o_ref[...] = jnp.maximum(prod + b_ref[...], 0.0)


_mm = pl.pallas_call(
    _mm_body,
    grid=(NPAD // MM_BLOCK,),
    in_specs=[
        pl.BlockSpec((MM_BLOCK, D), lambda i: (i, 0)),
        pl.BlockSpec((D, D), lambda i: (0, 0)),
        pl.BlockSpec((1, D), lambda i: (0, 0)),
    ],
    out_specs=pl.BlockSpec((MM_BLOCK, D), lambda i: (i, 0)),
    out_shape=jax.ShapeDtypeStruct((NPAD, D), jnp.float32),
)


def _sc_body(x_hbm, edge_hbm, out_hbm, idx_v, rows_v, out_v,
             sem0, sem1, sem2, sem3):
    sems = (sem0, sem1, sem2, sem3)
    wid = lax.axis_index("s") * NC + lax.axis_index("c")
    base = wid * NPW
    pltpu.sync_copy(edge_hbm.at[pl.ds(base * K, NPW * K)], idx_v)

    def _gather(g, slot):
        pltpu.async_copy(
            x_hbm.at[idx_v.at[pl.ds(g * GK, GK)]],
            rows_v.at[slot], sems[slot])

    for b in range(NBUF):
        _gather(b, b)

    @pl.loop(0, NG, step=NBUF)
    def _outer(g0):
        for b in range(NBUF):
            g = g0 + b
            pltpu.make_async_copy(
                x_hbm.at[idx_v.at[pl.ds(g * GK, GK)]],
                rows_v.at[b], sems[b]).wait()
            rows = rows_v.at[b]
            for gi in range(G):
                node = g * G + gi
                sls = [pl.ds(dv * LANES, LANES) for dv in range(DV)]
                accs = [rows[gi * K, sl] for sl in sls]
                for k in range(1, K):
                    r = gi * K + k
                    accs = [acc + rows[r, sl] for acc, sl in zip(accs, sls)]
                for dv in range(DV):
                    out_v[node, sls[dv]] = accs[dv] * (1.0 / K)
            gn = g + NBUF

            @pl.when(gn < NG)
            def _():
                _gather(gn, b)

    pltpu.sync_copy(out_v, out_hbm.at[pl.ds(base, NPW)])


_sc_gather_mean = pl.kernel(
    _sc_body,
    out_type=jax.ShapeDtypeStruct((S, D), jnp.float32),
    mesh=plsc.VectorSubcoreMesh(core_axis_name="c", subcore_axis_name="s"),
    scratch_types=[
        pltpu.VMEM((NPW * K,), jnp.int32),
        pltpu.VMEM((NBUF, GK, D), jnp.float32),
        pltpu.VMEM((NPW, D), jnp.float32),
        pltpu.SemaphoreType.DMA,
        pltpu.SemaphoreType.DMA,
        pltpu.SemaphoreType.DMA,
        pltpu.SemaphoreType.DMA,
    ],
)


def _tc_gather_body(e_ref, x_ref, o_ref):
    def node_body(n, carry):
        rows = [x_ref[pl.ds(e_ref[n, k], 1), :] for k in range(K)]
        while len(rows) > 1:
            rows = [rows[i] + rows[i + 1] for i in range(0, len(rows), 2)]
        o_ref[pl.ds(n, 1), :] = rows[0] * (1.0 / K)
        return carry

    lax.fori_loop(0, CHUNK, node_body, 0)


_tc_gather_mean = pl.pallas_call(
    _tc_gather_body,
    grid=(M // CHUNK,),
    in_specs=[
        pl.BlockSpec((CHUNK, K), lambda i: (i, 0),
                     memory_space=pltpu.SMEM),
        pl.BlockSpec((NPAD, D), lambda i: (0, 0)),
    ],
    out_specs=pl.BlockSpec((CHUNK, D), lambda i: (i, 0)),
    out_shape=jax.ShapeDtypeStruct((M, D), jnp.float32),
)


def kernel(idx, feats, edge_dict, sadj, epoch, W, b):
    feats_pad = jnp.concatenate(
        [feats, jnp.zeros((NPAD - N, D), jnp.float32)])
    x = _mm(feats_pad, W, b.reshape(1, D))
    edge = jnp.concatenate(
        [edge_dict.astype(jnp.int32),
         jnp.zeros((NPAD - N, K), jnp.int32)])
    sc_out = _sc_gather_mean(x, edge[:S].reshape(-1))
    tc_out = _tc_gather_mean(edge[S:], x)
    return jnp.concatenate([sc_out, tc_out])[:N]


# speedup vs baseline: 1.0560x; 1.0098x over previous
import jax
import jax.numpy as jnp
from jax import lax
from jax.experimental import pallas as pl
from jax.experimental.pallas import tpu as pltpu
from jax.experimental.pallas import tpu_sc as plsc

N, K, D = 10000, 32, 128

NC, NS = 2, 16
NW = NC * NS
NPAD = 10240
NPW = 232
S = NW * NPW
M = NPAD - S
G = 2
GK = G * K
NG = NPW // G
NBUF = 4
LANES = 16
DV = D // LANES

MM_BLOCK = 1024
CHUNK = 256


def _mm_body(f_ref, w_ref, b_ref, o_ref):
    prod = lax.dot_general(f_ref[...], w_ref[...], (((1,), (1,)), ((), ())),
                           preferred_element_type=jnp.float32)
    o_ref[...] = jnp.maximum(prod + b_ref[...], 0.0)


_mm = pl.pallas_call(
    _mm_body,
    grid=(NPAD // MM_BLOCK,),
    in_specs=[
        pl.BlockSpec((MM_BLOCK, D), lambda i: (i, 0)),
        pl.BlockSpec((D, D), lambda i: (0, 0)),
        pl.BlockSpec((1, D), lambda i: (0, 0)),
    ],
    out_specs=pl.BlockSpec((MM_BLOCK, D), lambda i: (i, 0)),
    out_shape=jax.ShapeDtypeStruct((NPAD, D), jnp.float32),
)


def _sc_body(x_hbm, edge_hbm, out_hbm, idx_v, rows_v, out_v,
             sem0, sem1, sem2, sem3):
    sems = (sem0, sem1, sem2, sem3)
    wid = lax.axis_index("s") * NC + lax.axis_index("c")
    base = wid * NPW
    pltpu.sync_copy(edge_hbm.at[pl.ds(base * K, NPW * K)], idx_v)

    def _gather(g, slot):
        pltpu.async_copy(
            x_hbm.at[idx_v.at[pl.ds(g * GK, GK)]],
            rows_v.at[slot], sems[slot])

    for b in range(NBUF):
        _gather(b, b)

    @pl.loop(0, NG, step=NBUF)
    def _outer(g0):
        for b in range(NBUF):
            g = g0 + b
            pltpu.make_async_copy(
                x_hbm.at[idx_v.at[pl.ds(g * GK, GK)]],
                rows_v.at[b], sems[b]).wait()
            rows = rows_v.at[b]
            for gi in range(G):
                node = g * G + gi
                sls = [pl.ds(dv * LANES, LANES) for dv in range(DV)]
                accs = [rows[gi * K, sl] for sl in sls]
                for k in range(1, K):
                    r = gi * K + k
                    accs = [acc + rows[r, sl] for acc, sl in zip(accs, sls)]
                for dv in range(DV):
                    out_v[node, sls[dv]] = accs[dv] * (1.0 / K)
            gn = g + NBUF

            @pl.when(gn < NG)
            def _():
                _gather(gn, b)

    pltpu.sync_copy(out_v, out_hbm.at[pl.ds(base, NPW)])


_sc_gather_mean = pl.kernel(
    _sc_body,
    out_type=jax.ShapeDtypeStruct((S, D), jnp.float32),
    mesh=plsc.VectorSubcoreMesh(core_axis_name="c", subcore_axis_name="s"),
    scratch_types=[
        pltpu.VMEM((NPW * K,), jnp.int32),
        pltpu.VMEM((NBUF, GK, D), jnp.float32),
        pltpu.VMEM((NPW, D), jnp.float32),
        pltpu.SemaphoreType.DMA,
        pltpu.SemaphoreType.DMA,
        pltpu.SemaphoreType.DMA,
        pltpu.SemaphoreType.DMA,
    ],
)


def _tc_gather_body(e_ref, x_ref, o_ref):
    def node_body(i, carry):
        for n in (2 * i, 2 * i + 1):
            rows = [x_ref[pl.ds(e_ref[n, k], 1), :] for k in range(K)]
            while len(rows) > 1:
                rows = [rows[j] + rows[j + 1]
                        for j in range(0, len(rows), 2)]
            o_ref[pl.ds(n, 1), :] = rows[0] * (1.0 / K)
        return carry

    lax.fori_loop(0, CHUNK // 2, node_body, 0)


_tc_gather_mean = pl.pallas_call(
    _tc_gather_body,
    grid=(M // CHUNK,),
    in_specs=[
        pl.BlockSpec((CHUNK, K), lambda i: (i, 0),
                     memory_space=pltpu.SMEM),
        pl.BlockSpec((NPAD, D), lambda i: (0, 0)),
    ],
    out_specs=pl.BlockSpec((CHUNK, D), lambda i: (i, 0)),
    out_shape=jax.ShapeDtypeStruct((M, D), jnp.float32),
)


def kernel(idx, feats, edge_dict, sadj, epoch, W, b):
    feats_pad = jnp.concatenate(
        [feats, jnp.zeros((NPAD - N, D), jnp.float32)])
    x = _mm(feats_pad, W, b.reshape(1, D))
    edge = jnp.concatenate(
        [edge_dict.astype(jnp.int32),
         jnp.zeros((NPAD - N, K), jnp.int32)])
    sc_out = _sc_gather_mean(x, edge[:S].reshape(-1))
    tc_out = _tc_gather_mean(edge[S:], x)
    return jnp.concatenate([sc_out, tc_out])[:N]


# speedup vs baseline: 1.0678x; 1.0112x over previous
import jax
import jax.numpy as jnp
from jax import lax
from jax.experimental import pallas as pl
from jax.experimental.pallas import tpu as pltpu
from jax.experimental.pallas import tpu_sc as plsc

N, K, D = 10000, 32, 128

NC, NS = 2, 16
NW = NC * NS
NPAD = 10240
NPW = 224
S = NW * NPW
M = NPAD - S
G = 2
GK = G * K
NG = NPW // G
NBUF = 4
LANES = 16
DV = D // LANES

MM_BLOCK = 1024
CHUNK = 256


def _mm_body(f_ref, w_ref, b_ref, o_ref):
    prod = lax.dot_general(f_ref[...], w_ref[...], (((1,), (1,)), ((), ())),
                           preferred_element_type=jnp.float32)
    o_ref[...] = jnp.maximum(prod + b_ref[...], 0.0)


_mm = pl.pallas_call(
    _mm_body,
    grid=(NPAD // MM_BLOCK,),
    in_specs=[
        pl.BlockSpec((MM_BLOCK, D), lambda i: (i, 0)),
        pl.BlockSpec((D, D), lambda i: (0, 0)),
        pl.BlockSpec((1, D), lambda i: (0, 0)),
    ],
    out_specs=pl.BlockSpec((MM_BLOCK, D), lambda i: (i, 0)),
    out_shape=jax.ShapeDtypeStruct((NPAD, D), jnp.float32),
)


def _sc_body(x_hbm, edge_hbm, out_hbm, idx_v, rows_v, out_v,
             sem0, sem1, sem2, sem3):
    sems = (sem0, sem1, sem2, sem3)
    wid = lax.axis_index("s") * NC + lax.axis_index("c")
    base = wid * NPW
    pltpu.sync_copy(edge_hbm.at[pl.ds(base * K, NPW * K)], idx_v)

    def _gather(g, slot):
        pltpu.async_copy(
            x_hbm.at[idx_v.at[pl.ds(g * GK, GK)]],
            rows_v.at[slot], sems[slot])

    for b in range(NBUF):
        _gather(b, b)

    @pl.loop(0, NG, step=NBUF)
    def _outer(g0):
        for b in range(NBUF):
            g = g0 + b
            pltpu.make_async_copy(
                x_hbm.at[idx_v.at[pl.ds(g * GK, GK)]],
                rows_v.at[b], sems[b]).wait()
            rows = rows_v.at[b]
            for gi in range(G):
                node = g * G + gi
                sls = [pl.ds(dv * LANES, LANES) for dv in range(DV)]
                accs = [rows[gi * K, sl] for sl in sls]
                for k in range(1, K):
                    r = gi * K + k
                    accs = [acc + rows[r, sl] for acc, sl in zip(accs, sls)]
                for dv in range(DV):
                    out_v[node, sls[dv]] = accs[dv] * (1.0 / K)
            gn = g + NBUF

            @pl.when(gn < NG)
            def _():
                _gather(gn, b)

    pltpu.sync_copy(out_v, out_hbm.at[pl.ds(base, NPW)])


_sc_gather_mean = pl.kernel(
    _sc_body,
    out_type=jax.ShapeDtypeStruct((S, D), jnp.float32),
    mesh=plsc.VectorSubcoreMesh(core_axis_name="c", subcore_axis_name="s"),
    scratch_types=[
        pltpu.VMEM((NPW * K,), jnp.int32),
        pltpu.VMEM((NBUF, GK, D), jnp.float32),
        pltpu.VMEM((NPW, D), jnp.float32),
        pltpu.SemaphoreType.DMA,
        pltpu.SemaphoreType.DMA,
        pltpu.SemaphoreType.DMA,
        pltpu.SemaphoreType.DMA,
    ],
)


def _tc_gather_body(e_ref, x_ref, o_ref):
    def node_body(i, carry):
        for n in (2 * i, 2 * i + 1):
            rows = [x_ref[pl.ds(e_ref[n, k], 1), :] for k in range(K)]
            while len(rows) > 1:
                rows = [rows[j] + rows[j + 1]
                        for j in range(0, len(rows), 2)]
            o_ref[pl.ds(n, 1), :] = rows[0] * (1.0 / K)
        return carry

    lax.fori_loop(0, CHUNK // 2, node_body, 0)


_tc_gather_mean = pl.pallas_call(
    _tc_gather_body,
    grid=(M // CHUNK,),
    in_specs=[
        pl.BlockSpec((CHUNK, K), lambda i: (i, 0),
                     memory_space=pltpu.SMEM),
        pl.BlockSpec((NPAD, D), lambda i: (0, 0)),
    ],
    out_specs=pl.BlockSpec((CHUNK, D), lambda i: (i, 0)),
    out_shape=jax.ShapeDtypeStruct((M, D), jnp.float32),
)


def kernel(idx, feats, edge_dict, sadj, epoch, W, b):
    feats_pad = jnp.concatenate(
        [feats, jnp.zeros((NPAD - N, D), jnp.float32)])
    x = _mm(feats_pad, W, b.reshape(1, D))
    edge = jnp.concatenate(
        [edge_dict.astype(jnp.int32),
         jnp.zeros((NPAD - N, K), jnp.int32)])
    sc_out = _sc_gather_mean(x, edge[:S].reshape(-1))
    tc_out = _tc_gather_mean(edge[S:], x)
    return jnp.concatenate([sc_out, tc_out])[:N]
